# baseline (speedup 1.0000x reference)
import functools

import jax
import jax.numpy as jnp
from jax import lax
from jax.experimental import pallas as pl
from jax.experimental.pallas import tpu as pltpu
from jax.experimental.pallas import tpu_sc as plsc

N_NODES = 10000
D = 128
NC = 2
NS = 16
NW = NC * NS
CHUNK = 128
CHUNKS_PER_W = 80
SRC_ROWS = 88
WCH = 16
NWIN = CHUNKS_PER_W // WCH
E_PAD = NW * CHUNKS_PER_W * CHUNK
AGG_ROWS = N_NODES + 112
ROWS_PER_TILE = AGG_ROWS // NS

_sc_mesh = plsc.VectorSubcoreMesh(core_axis_name="c", subcore_axis_name="s")


@functools.partial(
    pl.kernel,
    out_type=jax.ShapeDtypeStruct((NC, AGG_ROWS, D), jnp.float32),
    mesh=_sc_mesh,
    scratch_types=[
        pltpu.VMEM_SHARED((AGG_ROWS, D), jnp.float32),
        pltpu.VMEM((SRC_ROWS, CHUNK), jnp.int32),
        pltpu.VMEM((WCH, CHUNK), jnp.int32),
        pltpu.VMEM((CHUNK, D), jnp.float32),
        pltpu.VMEM((CHUNK, D), jnp.float32),
        pltpu.SemaphoreType.DMA,
        pltpu.SemaphoreType.DMA,
    ],
)
def _sc_aggregate(x_hbm, src_hbm, dst_hbm, zeros_hbm, agg_out,
                  agg_sh, src_v, dst_v, rows0, rows1, sem0, sem1):
    c = lax.axis_index("c")
    s = lax.axis_index("s")
    w = c * NS + s

    pltpu.sync_copy(zeros_hbm.at[pl.ds(s * ROWS_PER_TILE, ROWS_PER_TILE)],
                    agg_sh.at[pl.ds(s * ROWS_PER_TILE, ROWS_PER_TILE)])
    pltpu.sync_copy(src_hbm.at[w], src_v)
    pltpu.sync_copy(dst_hbm.at[w], dst_v)
    plsc.subcore_barrier()

    pltpu.async_copy(x_hbm.at[src_v.at[0]], rows0, sem0)

    def window(win, carry):
        pltpu.sync_copy(dst_hbm.at[w * NWIN + win], dst_v)

        def pair(p, c2):
            j = win * WCH + 2 * p
            pltpu.async_copy(x_hbm.at[src_v.at[j + 1]], rows1, sem1)
            pltpu.make_async_copy(x_hbm.at[src_v.at[0]], rows0, sem0).wait()
            pltpu.sync_copy(rows0, agg_sh.at[dst_v.at[2 * p]], add=True)
            pltpu.async_copy(x_hbm.at[src_v.at[j + 2]], rows0, sem0)
            pltpu.make_async_copy(x_hbm.at[src_v.at[0]], rows1, sem1).wait()
            pltpu.sync_copy(rows1, agg_sh.at[dst_v.at[2 * p + 1]], add=True)
            return c2

        lax.fori_loop(0, WCH // 2, pair, carry, unroll=False)
        return carry

    lax.fori_loop(0, NWIN, window, 0, unroll=False)
    pltpu.make_async_copy(x_hbm.at[src_v.at[0]], rows0, sem0).wait()

    plsc.subcore_barrier()
    pltpu.sync_copy(agg_sh.at[pl.ds(s * ROWS_PER_TILE, ROWS_PER_TILE)],
                    agg_out.at[c, pl.ds(s * ROWS_PER_TILE, ROWS_PER_TILE)])


def _tc_body(x_ref, a0_ref, a1_ref, wxt_ref, wat_ref, b_ref, o_ref):
    agg = a0_ref[0] + a1_ref[0]
    acc = jnp.dot(x_ref[...], wxt_ref[...],
                  preferred_element_type=jnp.float32,
                  precision=lax.Precision.HIGHEST)
    acc = acc + jnp.dot(agg, wat_ref[...],
                        preferred_element_type=jnp.float32,
                        precision=lax.Precision.HIGHEST)
    acc = acc + b_ref[...]
    acc = jnp.maximum(acc, 0.0)
    ss = jnp.sum(acc * acc, axis=1, keepdims=True)
    norm = jnp.maximum(jnp.sqrt(ss), 1e-12)
    o_ref[...] = acc / norm


BN = 1000


def _tc_dense(x, agg, wxt, wat, b2):
    return pl.pallas_call(
        _tc_body,
        grid=(N_NODES // BN,),
        in_specs=[
            pl.BlockSpec((BN, D), lambda i: (i, 0)),
            pl.BlockSpec((1, BN, D), lambda i: (0, i, 0)),
            pl.BlockSpec((1, BN, D), lambda i: (1, i, 0)),
            pl.BlockSpec((D, D), lambda i: (0, 0)),
            pl.BlockSpec((D, D), lambda i: (0, 0)),
            pl.BlockSpec((1, D), lambda i: (0, 0)),
        ],
        out_specs=pl.BlockSpec((BN, D), lambda i: (i, 0)),
        out_shape=jax.ShapeDtypeStruct((N_NODES, D), jnp.float32),
    )(x, agg, agg, wxt, wat, b2)


def kernel(x, edge_index, W, b):
    x = x.astype(jnp.float32)
    ei = edge_index.astype(jnp.int32)
    src, dst = ei[0], ei[1]
    e = src.shape[0]

    pad = E_PAD - e
    src_p = jnp.concatenate([src, jnp.zeros((pad,), jnp.int32)])
    dst_p = jnp.concatenate([dst, jnp.full((pad,), N_NODES, jnp.int32)])
    src3 = src_p.reshape(NW, CHUNKS_PER_W, CHUNK)
    src3 = jnp.concatenate(
        [src3, jnp.zeros((NW, SRC_ROWS - CHUNKS_PER_W, CHUNK), jnp.int32)],
        axis=1)
    dst3 = dst_p.reshape(NW * NWIN, WCH, CHUNK)
    zeros = jnp.zeros((AGG_ROWS, D), jnp.float32)

    agg = _sc_aggregate(x, src3, dst3, zeros)

    wxt = W[:, :D].T
    wat = W[:, D:].T
    b2 = b.reshape(1, D)
    return _tc_dense(x, agg, wxt, wat, b2)


# speedup vs baseline: 2.9803x; 2.9803x over previous
import functools

import jax
import jax.numpy as jnp
from jax import lax
from jax.experimental import pallas as pl
from jax.experimental.pallas import tpu as pltpu
from jax.experimental.pallas import tpu_sc as plsc

N_NODES = 10000
D = 128
DH = 64
NC = 2
NS = 16
CHUNK = 128
CHUNKS_PER_T = 160
SRC_ROWS = 168
WCH = 16
NWIN = CHUNKS_PER_T // WCH
E_PAD = NS * CHUNKS_PER_T * CHUNK
AGG_ROWS = N_NODES + 112
ROWS_PER_TILE = AGG_ROWS // NS

_sc_mesh = plsc.VectorSubcoreMesh(core_axis_name="c", subcore_axis_name="s")


@functools.partial(
    pl.kernel,
    out_type=jax.ShapeDtypeStruct((NC, AGG_ROWS, DH), jnp.float32),
    mesh=_sc_mesh,
    scratch_types=[
        pltpu.VMEM_SHARED((AGG_ROWS, DH), jnp.float32),
        pltpu.VMEM_SHARED((AGG_ROWS, DH), jnp.float32),
        pltpu.VMEM((SRC_ROWS, CHUNK), jnp.int32),
        pltpu.VMEM((WCH, CHUNK), jnp.int32),
        pltpu.VMEM((CHUNK, DH), jnp.float32),
        pltpu.VMEM((CHUNK, DH), jnp.float32),
        pltpu.SemaphoreType.DMA,
        pltpu.SemaphoreType.DMA,
    ],
    compiler_params=pltpu.CompilerParams(use_tc_tiling_on_sc=False),
)
def _sc_aggregate(xs_hbm, src_hbm, dst_hbm, zeros_hbm, agg_out,
                  x_sh, agg_sh, src_v, dst_v, rows0, rows1, sem0, sem1):
    c = lax.axis_index("c")
    s = lax.axis_index("s")

    pltpu.sync_copy(xs_hbm.at[c, pl.ds(s * ROWS_PER_TILE, ROWS_PER_TILE)],
                    x_sh.at[pl.ds(s * ROWS_PER_TILE, ROWS_PER_TILE)])
    pltpu.sync_copy(zeros_hbm.at[pl.ds(s * ROWS_PER_TILE, ROWS_PER_TILE)],
                    agg_sh.at[pl.ds(s * ROWS_PER_TILE, ROWS_PER_TILE)])
    pltpu.sync_copy(src_hbm.at[s], src_v)
    plsc.subcore_barrier()

    pltpu.async_copy(x_sh.at[src_v.at[0]], rows0, sem0)

    def window(win, carry):
        pltpu.sync_copy(dst_hbm.at[s * NWIN + win], dst_v)

        def pair(p, c2):
            j = win * WCH + 2 * p
            pltpu.async_copy(x_sh.at[src_v.at[j + 1]], rows1, sem1)
            pltpu.make_async_copy(x_sh.at[src_v.at[0]], rows0, sem0).wait()
            pltpu.sync_copy(rows0, agg_sh.at[dst_v.at[2 * p]], add=True)
            pltpu.async_copy(x_sh.at[src_v.at[j + 2]], rows0, sem0)
            pltpu.make_async_copy(x_sh.at[src_v.at[0]], rows1, sem1).wait()
            pltpu.sync_copy(rows1, agg_sh.at[dst_v.at[2 * p + 1]], add=True)
            return c2

        lax.fori_loop(0, WCH // 2, pair, carry, unroll=False)
        return carry

    lax.fori_loop(0, NWIN, window, 0, unroll=False)
    pltpu.make_async_copy(x_sh.at[src_v.at[0]], rows0, sem0).wait()

    plsc.subcore_barrier()
    pltpu.sync_copy(agg_sh.at[pl.ds(s * ROWS_PER_TILE, ROWS_PER_TILE)],
                    agg_out.at[c, pl.ds(s * ROWS_PER_TILE, ROWS_PER_TILE)])


def _tc_body(x_ref, a0_ref, a1_ref, wxt_ref, wa0t_ref, wa1t_ref, b_ref,
             o_ref):
    acc = jnp.dot(x_ref[...], wxt_ref[...],
                  preferred_element_type=jnp.float32,
                  precision=lax.Precision.HIGHEST)
    acc = acc + jnp.dot(a0_ref[0], wa0t_ref[...],
                        preferred_element_type=jnp.float32,
                        precision=lax.Precision.HIGHEST)
    acc = acc + jnp.dot(a1_ref[0], wa1t_ref[...],
                        preferred_element_type=jnp.float32,
                        precision=lax.Precision.HIGHEST)
    acc = acc + b_ref[...]
    acc = jnp.maximum(acc, 0.0)
    ss = jnp.sum(acc * acc, axis=1, keepdims=True)
    norm = jnp.maximum(jnp.sqrt(ss), 1e-12)
    o_ref[...] = acc / norm


BN = 1000


def _tc_dense(x, agg, wxt, wa0t, wa1t, b2):
    return pl.pallas_call(
        _tc_body,
        grid=(N_NODES // BN,),
        in_specs=[
            pl.BlockSpec((BN, D), lambda i: (i, 0)),
            pl.BlockSpec((1, BN, DH), lambda i: (0, i, 0)),
            pl.BlockSpec((1, BN, DH), lambda i: (1, i, 0)),
            pl.BlockSpec((D, D), lambda i: (0, 0)),
            pl.BlockSpec((DH, D), lambda i: (0, 0)),
            pl.BlockSpec((DH, D), lambda i: (0, 0)),
            pl.BlockSpec((1, D), lambda i: (0, 0)),
        ],
        out_specs=pl.BlockSpec((BN, D), lambda i: (i, 0)),
        out_shape=jax.ShapeDtypeStruct((N_NODES, D), jnp.float32),
    )(x, agg, agg, wxt, wa0t, wa1t, b2)


def kernel(x, edge_index, W, b):
    x = x.astype(jnp.float32)
    ei = edge_index.astype(jnp.int32)
    src, dst = ei[0], ei[1]
    e = src.shape[0]

    pad = E_PAD - e
    src_p = jnp.concatenate([src, jnp.zeros((pad,), jnp.int32)])
    dst_p = jnp.concatenate([dst, jnp.full((pad,), N_NODES, jnp.int32)])
    src3 = src_p.reshape(NS, CHUNKS_PER_T, CHUNK)
    src3 = jnp.concatenate(
        [src3, jnp.zeros((NS, SRC_ROWS - CHUNKS_PER_T, CHUNK), jnp.int32)],
        axis=1)
    dst3 = dst_p.reshape(NS * NWIN, WCH, CHUNK)
    zeros = jnp.zeros((AGG_ROWS, DH), jnp.float32)

    x_pad = jnp.concatenate(
        [x, jnp.zeros((AGG_ROWS - N_NODES, D), jnp.float32)])
    xs = x_pad.reshape(AGG_ROWS, NC, DH).transpose(1, 0, 2)

    agg = _sc_aggregate(xs, src3, dst3, zeros)

    wxt = W[:, :D].T
    wa0t = W[:, D:D + DH].T
    wa1t = W[:, D + DH:].T
    b2 = b.reshape(1, D)
    return _tc_dense(x, agg, wxt, wa0t, wa1t, b2)


# speedup vs baseline: 3.2886x; 1.1034x over previous
import functools

import jax
import jax.numpy as jnp
from jax import lax
from jax.experimental import pallas as pl
from jax.experimental.pallas import tpu as pltpu
from jax.experimental.pallas import tpu_sc as plsc

N_NODES = 10000
D = 128
DH = 64
NC = 2
NS = 16
CHUNK = 125
CHUNKS_PER_T = 160
WCH = 16
NWIN = CHUNKS_PER_T // WCH
ROWS_PER_TILE = N_NODES // NS

_sc_mesh = plsc.VectorSubcoreMesh(core_axis_name="c", subcore_axis_name="s")


@functools.partial(
    pl.kernel,
    out_type=jax.ShapeDtypeStruct((NC, N_NODES, DH), jnp.float32),
    mesh=_sc_mesh,
    scratch_types=[
        pltpu.VMEM_SHARED((N_NODES, DH), jnp.float32),
        pltpu.VMEM_SHARED((N_NODES, DH), jnp.float32),
        pltpu.VMEM((CHUNKS_PER_T, CHUNK), jnp.int32),
        pltpu.VMEM((WCH, CHUNK), jnp.int32),
        pltpu.VMEM((CHUNK, DH), jnp.float32),
        pltpu.VMEM((CHUNK, DH), jnp.float32),
        pltpu.SemaphoreType.DMA,
        pltpu.SemaphoreType.DMA,
    ],
    compiler_params=pltpu.CompilerParams(use_tc_tiling_on_sc=False),
)
def _sc_aggregate(x_hbm, src_hbm, dst_hbm, zeros_hbm, agg_out,
                  x_sh, agg_sh, src_v, dst_v, rows0, rows1, sem0, sem1):
    c = lax.axis_index("c")
    s = lax.axis_index("s")

    pltpu.sync_copy(
        x_hbm.at[pl.ds(s * ROWS_PER_TILE, ROWS_PER_TILE), pl.ds(c * DH, DH)],
        x_sh.at[pl.ds(s * ROWS_PER_TILE, ROWS_PER_TILE)])
    pltpu.sync_copy(zeros_hbm.at[pl.ds(s * ROWS_PER_TILE, ROWS_PER_TILE)],
                    agg_sh.at[pl.ds(s * ROWS_PER_TILE, ROWS_PER_TILE)])
    pltpu.sync_copy(src_hbm.at[s], src_v)
    plsc.subcore_barrier()

    pltpu.async_copy(x_sh.at[src_v.at[0]], rows0, sem0)

    def window(win, carry):
        pltpu.sync_copy(dst_hbm.at[s * NWIN + win], dst_v)

        def pair(p, c2):
            j = win * WCH + 2 * p
            pltpu.async_copy(x_sh.at[src_v.at[j + 1]], rows1, sem1)
            pltpu.make_async_copy(x_sh.at[src_v.at[0]], rows0, sem0).wait()
            pltpu.sync_copy(rows0, agg_sh.at[dst_v.at[2 * p]], add=True)

            @pl.when(j + 2 < CHUNKS_PER_T)
            def _():
                pltpu.async_copy(x_sh.at[src_v.at[j + 2]], rows0, sem0)

            pltpu.make_async_copy(x_sh.at[src_v.at[0]], rows1, sem1).wait()
            pltpu.sync_copy(rows1, agg_sh.at[dst_v.at[2 * p + 1]], add=True)
            return c2

        lax.fori_loop(0, WCH // 2, pair, carry, unroll=False)
        return carry

    lax.fori_loop(0, NWIN, window, 0, unroll=False)

    plsc.subcore_barrier()
    pltpu.sync_copy(agg_sh.at[pl.ds(s * ROWS_PER_TILE, ROWS_PER_TILE)],
                    agg_out.at[c, pl.ds(s * ROWS_PER_TILE, ROWS_PER_TILE)])


def _tc_body(x_ref, a0_ref, a1_ref, wxt_ref, wa0t_ref, wa1t_ref, b_ref,
             o_ref):
    acc = jnp.dot(x_ref[...], wxt_ref[...],
                  preferred_element_type=jnp.float32,
                  precision=lax.Precision.HIGHEST)
    acc = acc + jnp.dot(a0_ref[0], wa0t_ref[...],
                        preferred_element_type=jnp.float32,
                        precision=lax.Precision.HIGHEST)
    acc = acc + jnp.dot(a1_ref[0], wa1t_ref[...],
                        preferred_element_type=jnp.float32,
                        precision=lax.Precision.HIGHEST)
    acc = acc + b_ref[...]
    acc = jnp.maximum(acc, 0.0)
    ss = jnp.sum(acc * acc, axis=1, keepdims=True)
    norm = jnp.maximum(jnp.sqrt(ss), 1e-12)
    o_ref[...] = acc / norm


BN = 1000


def _tc_dense(x, agg, wxt, wa0t, wa1t, b2):
    return pl.pallas_call(
        _tc_body,
        grid=(N_NODES // BN,),
        in_specs=[
            pl.BlockSpec((BN, D), lambda i: (i, 0)),
            pl.BlockSpec((1, BN, DH), lambda i: (0, i, 0)),
            pl.BlockSpec((1, BN, DH), lambda i: (1, i, 0)),
            pl.BlockSpec((D, D), lambda i: (0, 0)),
            pl.BlockSpec((DH, D), lambda i: (0, 0)),
            pl.BlockSpec((DH, D), lambda i: (0, 0)),
            pl.BlockSpec((1, D), lambda i: (0, 0)),
        ],
        out_specs=pl.BlockSpec((BN, D), lambda i: (i, 0)),
        out_shape=jax.ShapeDtypeStruct((N_NODES, D), jnp.float32),
    )(x, agg, agg, wxt, wa0t, wa1t, b2)


def kernel(x, edge_index, W, b):
    x = x.astype(jnp.float32)
    ei = edge_index.astype(jnp.int32)
    src3 = ei[0].reshape(NS, CHUNKS_PER_T, CHUNK)
    dst3 = ei[1].reshape(NS * NWIN, WCH, CHUNK)
    zeros = jnp.zeros((N_NODES, DH), jnp.float32)

    agg = _sc_aggregate(x, src3, dst3, zeros)

    wxt = W[:, :D].T
    wa0t = W[:, D:D + DH].T
    wa1t = W[:, D + DH:].T
    b2 = b.reshape(1, D)
    return _tc_dense(x, agg, wxt, wa0t, wa1t, b2)


# speedup vs baseline: 3.7489x; 1.1400x over previous
import functools

import jax
import jax.numpy as jnp
from jax import lax
from jax.experimental import pallas as pl
from jax.experimental.pallas import tpu as pltpu
from jax.experimental.pallas import tpu_sc as plsc

N_NODES = 10000
D = 128
DH = 64
NC = 2
NS = 16
CHUNK = 125
CHUNKS_PER_T = 160
WCH = 16
NWIN = CHUNKS_PER_T // WCH
ROWS_PER_TILE = N_NODES // NS
DST_BASE = NS * CHUNKS_PER_T

_sc_mesh = plsc.VectorSubcoreMesh(core_axis_name="c", subcore_axis_name="s")


@functools.partial(
    pl.kernel,
    out_type=jax.ShapeDtypeStruct((N_NODES, D), jnp.float32),
    mesh=_sc_mesh,
    scratch_types=[
        pltpu.VMEM_SHARED((N_NODES, DH), jnp.float32),
        pltpu.VMEM_SHARED((N_NODES, DH), jnp.float32),
        pltpu.VMEM((CHUNKS_PER_T, CHUNK), jnp.int32),
        pltpu.VMEM((WCH, CHUNK), jnp.int32),
        pltpu.VMEM((CHUNK, DH), jnp.float32),
        pltpu.VMEM((CHUNK, DH), jnp.float32),
        pltpu.SemaphoreType.DMA,
        pltpu.SemaphoreType.DMA,
    ],
    compiler_params=pltpu.CompilerParams(use_tc_tiling_on_sc=False),
)
def _sc_aggregate(x_hbm, ei_hbm, zeros_hbm, agg_out,
                  x_sh, agg_sh, src_v, dst_v, rows0, rows1, sem0, sem1):
    c = lax.axis_index("c")
    s = lax.axis_index("s")

    pltpu.sync_copy(
        x_hbm.at[pl.ds(s * ROWS_PER_TILE, ROWS_PER_TILE), pl.ds(c * DH, DH)],
        x_sh.at[pl.ds(s * ROWS_PER_TILE, ROWS_PER_TILE)])
    pltpu.sync_copy(zeros_hbm.at[pl.ds(s * ROWS_PER_TILE, ROWS_PER_TILE)],
                    agg_sh.at[pl.ds(s * ROWS_PER_TILE, ROWS_PER_TILE)])
    pltpu.sync_copy(ei_hbm.at[pl.ds(s * CHUNKS_PER_T, CHUNKS_PER_T)], src_v)
    plsc.subcore_barrier()

    pltpu.async_copy(x_sh.at[src_v.at[0]], rows0, sem0)

    def window(win, carry):
        pltpu.sync_copy(
            ei_hbm.at[pl.ds(DST_BASE + s * CHUNKS_PER_T + win * WCH, WCH)],
            dst_v)

        def pair(p, c2):
            j = win * WCH + 2 * p
            pltpu.async_copy(x_sh.at[src_v.at[j + 1]], rows1, sem1)
            pltpu.make_async_copy(x_sh.at[src_v.at[0]], rows0, sem0).wait()
            pltpu.sync_copy(rows0, agg_sh.at[dst_v.at[2 * p]], add=True)

            @pl.when(j + 2 < CHUNKS_PER_T)
            def _():
                pltpu.async_copy(x_sh.at[src_v.at[j + 2]], rows0, sem0)

            pltpu.make_async_copy(x_sh.at[src_v.at[0]], rows1, sem1).wait()
            pltpu.sync_copy(rows1, agg_sh.at[dst_v.at[2 * p + 1]], add=True)
            return c2

        lax.fori_loop(0, WCH // 2, pair, carry, unroll=False)
        return carry

    lax.fori_loop(0, NWIN, window, 0, unroll=False)

    plsc.subcore_barrier()
    pltpu.sync_copy(
        agg_sh.at[pl.ds(s * ROWS_PER_TILE, ROWS_PER_TILE)],
        agg_out.at[pl.ds(s * ROWS_PER_TILE, ROWS_PER_TILE), pl.ds(c * DH, DH)])


def _tc_body(x_ref, a_ref, wxt_ref, wat_ref, b_ref, o_ref):
    acc = jnp.dot(x_ref[...], wxt_ref[...],
                  preferred_element_type=jnp.float32,
                  precision=lax.Precision.HIGHEST)
    acc = acc + jnp.dot(a_ref[...], wat_ref[...],
                        preferred_element_type=jnp.float32,
                        precision=lax.Precision.HIGHEST)
    acc = acc + b_ref[...]
    acc = jnp.maximum(acc, 0.0)
    ss = jnp.sum(acc * acc, axis=1, keepdims=True)
    norm = jnp.maximum(jnp.sqrt(ss), 1e-12)
    o_ref[...] = acc / norm


BN = 1000


def _tc_dense(x, agg, wxt, wat, b2):
    return pl.pallas_call(
        _tc_body,
        grid=(N_NODES // BN,),
        in_specs=[
            pl.BlockSpec((BN, D), lambda i: (i, 0)),
            pl.BlockSpec((BN, D), lambda i: (i, 0)),
            pl.BlockSpec((D, D), lambda i: (0, 0)),
            pl.BlockSpec((D, D), lambda i: (0, 0)),
            pl.BlockSpec((1, D), lambda i: (0, 0)),
        ],
        out_specs=pl.BlockSpec((BN, D), lambda i: (i, 0)),
        out_shape=jax.ShapeDtypeStruct((N_NODES, D), jnp.float32),
    )(x, agg, wxt, wat, b2)


def kernel(x, edge_index, W, b):
    x = x.astype(jnp.float32)
    ei2 = edge_index.astype(jnp.int32).reshape(2 * NS * CHUNKS_PER_T, CHUNK)
    zeros = jnp.zeros((N_NODES, DH), jnp.float32)

    agg = _sc_aggregate(x, ei2, zeros)

    wxt = W[:, :D].T
    wat = W[:, D:].T
    b2 = b.reshape(1, D)
    return _tc_dense(x, agg, wxt, wat, b2)


# speedup vs baseline: 3.7904x; 1.0111x over previous
import functools

import jax
import jax.numpy as jnp
from jax import lax
from jax.experimental import pallas as pl
from jax.experimental.pallas import tpu as pltpu
from jax.experimental.pallas import tpu_sc as plsc

N_NODES = 10000
D = 128
DH = 64
NC = 2
NS = 16
CHUNK = 200
CHUNKS_PER_T = 100
WCH = 10
NWIN = CHUNKS_PER_T // WCH
ROWS_PER_TILE = N_NODES // NS
DST_BASE = NS * CHUNKS_PER_T

_sc_mesh = plsc.VectorSubcoreMesh(core_axis_name="c", subcore_axis_name="s")


@functools.partial(
    pl.kernel,
    out_type=jax.ShapeDtypeStruct((N_NODES, D), jnp.float32),
    mesh=_sc_mesh,
    scratch_types=[
        pltpu.VMEM_SHARED((N_NODES, DH), jnp.float32),
        pltpu.VMEM_SHARED((N_NODES, DH), jnp.float32),
        pltpu.VMEM((CHUNKS_PER_T, CHUNK), jnp.int32),
        pltpu.VMEM((WCH, CHUNK), jnp.int32),
        pltpu.VMEM((CHUNK, DH), jnp.float32),
        pltpu.VMEM((CHUNK, DH), jnp.float32),
        pltpu.SemaphoreType.DMA,
        pltpu.SemaphoreType.DMA,
    ],
    compiler_params=pltpu.CompilerParams(use_tc_tiling_on_sc=False),
)
def _sc_aggregate(x_hbm, ei_hbm, zeros_hbm, agg_out,
                  x_sh, agg_sh, src_v, dst_v, rows0, rows1, sem0, sem1):
    c = lax.axis_index("c")
    s = lax.axis_index("s")

    pltpu.sync_copy(
        x_hbm.at[pl.ds(s * ROWS_PER_TILE, ROWS_PER_TILE), pl.ds(c * DH, DH)],
        x_sh.at[pl.ds(s * ROWS_PER_TILE, ROWS_PER_TILE)])
    pltpu.sync_copy(zeros_hbm.at[pl.ds(s * ROWS_PER_TILE, ROWS_PER_TILE)],
                    agg_sh.at[pl.ds(s * ROWS_PER_TILE, ROWS_PER_TILE)])
    pltpu.sync_copy(ei_hbm.at[pl.ds(s * CHUNKS_PER_T, CHUNKS_PER_T)], src_v)
    plsc.subcore_barrier()

    pltpu.async_copy(x_sh.at[src_v.at[0]], rows0, sem0)

    def window(win, carry):
        pltpu.sync_copy(
            ei_hbm.at[pl.ds(DST_BASE + s * CHUNKS_PER_T + win * WCH, WCH)],
            dst_v)

        def pair(p, c2):
            j = win * WCH + 2 * p
            pltpu.async_copy(x_sh.at[src_v.at[j + 1]], rows1, sem1)
            pltpu.make_async_copy(x_sh.at[src_v.at[0]], rows0, sem0).wait()
            pltpu.sync_copy(rows0, agg_sh.at[dst_v.at[2 * p]], add=True)

            @pl.when(j + 2 < CHUNKS_PER_T)
            def _():
                pltpu.async_copy(x_sh.at[src_v.at[j + 2]], rows0, sem0)

            pltpu.make_async_copy(x_sh.at[src_v.at[0]], rows1, sem1).wait()
            pltpu.sync_copy(rows1, agg_sh.at[dst_v.at[2 * p + 1]], add=True)
            return c2

        lax.fori_loop(0, WCH // 2, pair, carry, unroll=False)
        return carry

    lax.fori_loop(0, NWIN, window, 0, unroll=False)

    plsc.subcore_barrier()
    pltpu.sync_copy(
        agg_sh.at[pl.ds(s * ROWS_PER_TILE, ROWS_PER_TILE)],
        agg_out.at[pl.ds(s * ROWS_PER_TILE, ROWS_PER_TILE), pl.ds(c * DH, DH)])


def _tc_body(x_ref, a_ref, wxt_ref, wat_ref, b_ref, o_ref):
    acc = jnp.dot(x_ref[...], wxt_ref[...],
                  preferred_element_type=jnp.float32,
                  precision=lax.Precision.HIGHEST)
    acc = acc + jnp.dot(a_ref[...], wat_ref[...],
                        preferred_element_type=jnp.float32,
                        precision=lax.Precision.HIGHEST)
    acc = acc + b_ref[...]
    acc = jnp.maximum(acc, 0.0)
    ss = jnp.sum(acc * acc, axis=1, keepdims=True)
    norm = jnp.maximum(jnp.sqrt(ss), 1e-12)
    o_ref[...] = acc / norm


BN = 1000


def _tc_dense(x, agg, wxt, wat, b2):
    return pl.pallas_call(
        _tc_body,
        grid=(N_NODES // BN,),
        in_specs=[
            pl.BlockSpec((BN, D), lambda i: (i, 0)),
            pl.BlockSpec((BN, D), lambda i: (i, 0)),
            pl.BlockSpec((D, D), lambda i: (0, 0)),
            pl.BlockSpec((D, D), lambda i: (0, 0)),
            pl.BlockSpec((1, D), lambda i: (0, 0)),
        ],
        out_specs=pl.BlockSpec((BN, D), lambda i: (i, 0)),
        out_shape=jax.ShapeDtypeStruct((N_NODES, D), jnp.float32),
    )(x, agg, wxt, wat, b2)


def kernel(x, edge_index, W, b):
    x = x.astype(jnp.float32)
    ei2 = edge_index.astype(jnp.int32).reshape(2 * NS * CHUNKS_PER_T, CHUNK)
    zeros = jnp.zeros((N_NODES, DH), jnp.float32)

    agg = _sc_aggregate(x, ei2, zeros)

    wxt = W[:, :D].T
    wat = W[:, D:].T
    b2 = b.reshape(1, D)
    return _tc_dense(x, agg, wxt, wat, b2)


# speedup vs baseline: 3.9905x; 1.0528x over previous
import functools

import jax
import jax.numpy as jnp
from jax import lax
from jax.experimental import pallas as pl
from jax.experimental.pallas import tpu as pltpu
from jax.experimental.pallas import tpu_sc as plsc

N_NODES = 10000
D = 128
DH = 64
NC = 2
NS = 16
CHUNK = 200
CHUNKS_PER_T = 100
WCH = 10
NWIN = CHUNKS_PER_T // WCH
ROWS_PER_TILE = N_NODES // NS
DST_BASE = NS * CHUNKS_PER_T

_sc_mesh = plsc.VectorSubcoreMesh(core_axis_name="c", subcore_axis_name="s")


@functools.partial(
    pl.kernel,
    out_type=jax.ShapeDtypeStruct((N_NODES, D), jnp.float32),
    mesh=_sc_mesh,
    scratch_types=[
        pltpu.VMEM_SHARED((N_NODES, DH), jnp.float32),
        pltpu.VMEM_SHARED((N_NODES, DH), jnp.float32),
        pltpu.VMEM((CHUNKS_PER_T, CHUNK), jnp.int32),
        pltpu.VMEM((WCH, CHUNK), jnp.int32),
        pltpu.VMEM((CHUNK, DH), jnp.float32),
        pltpu.VMEM((CHUNK, DH), jnp.float32),
        pltpu.SemaphoreType.DMA,
        pltpu.SemaphoreType.DMA,
    ],
    compiler_params=pltpu.CompilerParams(use_tc_tiling_on_sc=False),
)
def _sc_aggregate(x_hbm, ei_hbm, zeros_hbm, agg_out,
                  x_sh, agg_sh, src_v, dst_v, rows0, rows1, sem0, sem1):
    c = lax.axis_index("c")
    s = lax.axis_index("s")

    pltpu.sync_copy(
        x_hbm.at[pl.ds(s * ROWS_PER_TILE, ROWS_PER_TILE), pl.ds(c * DH, DH)],
        x_sh.at[pl.ds(s * ROWS_PER_TILE, ROWS_PER_TILE)])
    pltpu.sync_copy(zeros_hbm.at[pl.ds(s * ROWS_PER_TILE, ROWS_PER_TILE)],
                    agg_sh.at[pl.ds(s * ROWS_PER_TILE, ROWS_PER_TILE)])
    pltpu.sync_copy(ei_hbm.at[pl.ds(s * CHUNKS_PER_T, CHUNKS_PER_T)], src_v)
    plsc.subcore_barrier()

    pltpu.async_copy(x_sh.at[src_v.at[0]], rows0, sem0)

    def window(win, carry):
        pltpu.sync_copy(
            ei_hbm.at[pl.ds(DST_BASE + s * CHUNKS_PER_T + win * WCH, WCH)],
            dst_v)

        def pair(p, c2):
            j = win * WCH + 2 * p
            pltpu.async_copy(x_sh.at[src_v.at[j + 1]], rows1, sem1)
            pltpu.make_async_copy(x_sh.at[src_v.at[0]], rows0, sem0).wait()
            pltpu.sync_copy(rows0, agg_sh.at[dst_v.at[2 * p]], add=True)

            @pl.when(j + 2 < CHUNKS_PER_T)
            def _():
                pltpu.async_copy(x_sh.at[src_v.at[j + 2]], rows0, sem0)

            pltpu.make_async_copy(x_sh.at[src_v.at[0]], rows1, sem1).wait()
            pltpu.sync_copy(rows1, agg_sh.at[dst_v.at[2 * p + 1]], add=True)
            return c2

        lax.fori_loop(0, WCH // 2, pair, carry, unroll=False)
        return carry

    lax.fori_loop(0, NWIN, window, 0, unroll=False)

    plsc.subcore_barrier()
    pltpu.sync_copy(
        agg_sh.at[pl.ds(s * ROWS_PER_TILE, ROWS_PER_TILE)],
        agg_out.at[pl.ds(s * ROWS_PER_TILE, ROWS_PER_TILE), pl.ds(c * DH, DH)])


def _tc_body(x_ref, a_ref, wxt_ref, wat_ref, b_ref, o_ref):
    acc = jnp.dot(x_ref[...], wxt_ref[...],
                  preferred_element_type=jnp.float32)
    acc = acc + jnp.dot(a_ref[...], wat_ref[...],
                        preferred_element_type=jnp.float32)
    acc = acc + b_ref[...]
    acc = jnp.maximum(acc, 0.0)
    ss = jnp.sum(acc * acc, axis=1, keepdims=True)
    o_ref[...] = acc * lax.rsqrt(jnp.maximum(ss, 1e-24))


BN = 1000


def _tc_dense(x, agg, wxt, wat, b2):
    return pl.pallas_call(
        _tc_body,
        grid=(N_NODES // BN,),
        in_specs=[
            pl.BlockSpec((BN, D), lambda i: (i, 0)),
            pl.BlockSpec((BN, D), lambda i: (i, 0)),
            pl.BlockSpec((D, D), lambda i: (0, 0)),
            pl.BlockSpec((D, D), lambda i: (0, 0)),
            pl.BlockSpec((1, D), lambda i: (0, 0)),
        ],
        out_specs=pl.BlockSpec((BN, D), lambda i: (i, 0)),
        out_shape=jax.ShapeDtypeStruct((N_NODES, D), jnp.float32),
    )(x, agg, wxt, wat, b2)


def kernel(x, edge_index, W, b):
    x = x.astype(jnp.float32)
    ei2 = edge_index.astype(jnp.int32).reshape(2 * NS * CHUNKS_PER_T, CHUNK)
    zeros = jnp.zeros((N_NODES, DH), jnp.float32)

    agg = _sc_aggregate(x, ei2, zeros)

    wxt = W[:, :D].T
    wat = W[:, D:].T
    b2 = b.reshape(1, D)
    return _tc_dense(x, agg, wxt, wat, b2)
